# Initial kernel scaffold; baseline (speedup 1.0000x reference)
import jax
import jax.numpy as jnp
from jax.experimental import pallas as pl


def kernel(x, edge_index, W1l, W1r, b1, W2l, W2r, b2):
    raise NotImplementedError("write your pallas kernel here")


# speedup vs baseline: 6.0503x; 6.0503x over previous
import functools

import jax
import jax.numpy as jnp
from jax import lax
from jax.experimental import pallas as pl
from jax.experimental.pallas import tpu as pltpu
from jax.experimental.pallas import tpu_sc as plsc

N = 10000
E = 320000
D_IN = 128
D_H = 40
D_OUT = 24

NPAD = 10240
D1 = 48
D2 = 32
NUM_TILES = 32
EPT = E // NUM_TILES
CHUNK = 80
NCHUNK = EPT // CHUNK
ROWS_PER_TILE = NPAD // 16


@functools.lru_cache(maxsize=None)
def _make_sc_scatter(width):
    mesh = plsc.VectorSubcoreMesh(core_axis_name="c", subcore_axis_name="s")

    @functools.partial(
        pl.kernel,
        out_type=jax.ShapeDtypeStruct((2, NPAD, width), jnp.float32),
        mesh=mesh,
        scratch_types=[
            pltpu.VMEM((CHUNK,), jnp.int32),
            pltpu.VMEM((CHUNK,), jnp.int32),
            pltpu.VMEM((CHUNK, width), jnp.float32),
            pltpu.VMEM((128, width), jnp.float32),
            pltpu.VMEM_SHARED((NPAD, width), jnp.float32),
            pltpu.SemaphoreType.DMA,
        ],
        compiler_params=pltpu.CompilerParams(use_tc_tiling_on_sc=False),
    )
    def sc_scatter(table_hbm, src_hbm, dst_hbm, out_hbm,
                   sidx, didx, rows, zblk, acc, sem):
        cid = lax.axis_index("c")
        sid = lax.axis_index("s")

        @pl.loop(0, 128)
        def _zero_rows(i):
            @pl.loop(0, width // 16)
            def _zero_lanes(j):
                zblk[i, pl.ds(j * 16, 16)] = jnp.zeros((16,), jnp.float32)

        @pl.loop(0, ROWS_PER_TILE // 128)
        def _zero_acc(r):
            pltpu.sync_copy(zblk, acc.at[pl.ds(sid * ROWS_PER_TILE + r * 128, 128)])

        plsc.subcore_barrier()

        base = (cid * 16 + sid) * EPT

        @pl.loop(0, NCHUNK)
        def _edges(ch):
            off = base + ch * CHUNK
            pltpu.sync_copy(src_hbm.at[pl.ds(off, CHUNK)], sidx)
            pltpu.sync_copy(dst_hbm.at[pl.ds(off, CHUNK)], didx)
            pltpu.async_copy(table_hbm.at[sidx], rows, sem).wait()
            pltpu.sync_copy(rows, acc.at[didx], add=True)

        plsc.subcore_barrier()

        @pl.loop(0, ROWS_PER_TILE // 128)
        def _writeout(r):
            row0 = sid * ROWS_PER_TILE + r * 128
            pltpu.sync_copy(acc.at[pl.ds(row0, 128)],
                            out_hbm.at[cid, pl.ds(row0, 128)])

    return sc_scatter


_BR = 1000


def _dot_t(a, w):
    return lax.dot_general(a, w, (((1,), (1,)), ((), ())),
                           preferred_element_type=jnp.float32)


def _tc_pre_body(x_ref, w1l_ref, w1r_ref, b1_ref, table_ref, p1_ref):
    x = x_ref[...]
    y = _dot_t(x, w1l_ref[...])
    col = lax.broadcasted_iota(jnp.int32, (_BR, D1), 1)
    table_ref[...] = y + jnp.where(col == D_H, 1.0, 0.0)
    p1_ref[...] = _dot_t(x, w1r_ref[...]) + b1_ref[...]


def _tc_mid_body(a0_ref, a1_ref, p1_ref, w2l_ref, w2r_ref, b2_ref,
                 table2_ref, p2_ref, dinv_ref):
    acc = a0_ref[...] + a1_ref[...]
    deg = acc[:, D_H:D_H + 1]
    dinv = 1.0 / jnp.maximum(deg, 1.0)
    h = jnp.maximum(acc[:, :D_H] * dinv + p1_ref[...], 0.0)
    table2_ref[...] = _dot_t(h, w2l_ref[...])
    p2_ref[...] = _dot_t(h, w2r_ref[...]) + b2_ref[...]
    dinv_ref[...] = dinv


def _tc_post_body(a0_ref, a1_ref, dinv_ref, p2_ref, out_ref):
    agg = (a0_ref[...] + a1_ref[...])[:, :D_OUT] * dinv_ref[...]
    z = agg + p2_ref[...]
    m = jnp.max(z, axis=1, keepdims=True)
    zs = z - m
    out_ref[...] = zs - jnp.log(jnp.sum(jnp.exp(zs), axis=1, keepdims=True))


def _row_spec(w):
    return pl.BlockSpec((_BR, w), lambda i: (i, 0))


def _full_spec(shape):
    return pl.BlockSpec(shape, lambda i: tuple(0 for _ in shape))


def kernel(x, edge_index, W1l, W1r, b1, W2l, W2r, b2):
    src = edge_index[0]
    dst = edge_index[1]

    w1l_pad = jnp.zeros((D1, D_IN), jnp.float32).at[:D_H].set(W1l)
    w2l_pad = jnp.zeros((D2, D_H), jnp.float32).at[:D_OUT].set(W2l)
    b1r = b1.reshape(1, D_H)
    b2r = b2.reshape(1, D_OUT)

    grid = (N // _BR,)

    table1, p1 = pl.pallas_call(
        _tc_pre_body,
        grid=grid,
        in_specs=[_row_spec(D_IN), _full_spec((D1, D_IN)),
                  _full_spec((D_H, D_IN)), _full_spec((1, D_H))],
        out_specs=[_row_spec(D1), _row_spec(D_H)],
        out_shape=[jax.ShapeDtypeStruct((N, D1), jnp.float32),
                   jax.ShapeDtypeStruct((N, D_H), jnp.float32)],
    )(x, w1l_pad, W1r, b1r)

    part1 = _make_sc_scatter(D1)(table1, src, dst)

    table2, p2, dinv = pl.pallas_call(
        _tc_mid_body,
        grid=grid,
        in_specs=[_row_spec(D1), _row_spec(D1), _row_spec(D_H),
                  _full_spec((D2, D_H)), _full_spec((D_OUT, D_H)),
                  _full_spec((1, D_OUT))],
        out_specs=[_row_spec(D2), _row_spec(D_OUT), _row_spec(1)],
        out_shape=[jax.ShapeDtypeStruct((N, D2), jnp.float32),
                   jax.ShapeDtypeStruct((N, D_OUT), jnp.float32),
                   jax.ShapeDtypeStruct((N, 1), jnp.float32)],
    )(part1[0], part1[1], p1, w2l_pad, W2r, b2r)

    part2 = _make_sc_scatter(D2)(table2, src, dst)

    out = pl.pallas_call(
        _tc_post_body,
        grid=grid,
        in_specs=[_row_spec(D2), _row_spec(D2), _row_spec(1),
                  _row_spec(D_OUT)],
        out_specs=_row_spec(D_OUT),
        out_shape=jax.ShapeDtypeStruct((N, D_OUT), jnp.float32),
    )(part2[0], part2[1], dinv, p2)

    return out


# speedup vs baseline: 6.9579x; 1.1500x over previous
import functools

import jax
import jax.numpy as jnp
from jax import lax
from jax.experimental import pallas as pl
from jax.experimental.pallas import tpu as pltpu
from jax.experimental.pallas import tpu_sc as plsc

N = 10000
E = 320000
D_IN = 128
D_H = 40
D_OUT = 24

NPAD = 10240
D1 = 48
D2 = 32
NUM_TILES = 32
CHUNK = 128
NCHUNK = 80
EPT = CHUNK * NCHUNK
E_PAD = EPT * NUM_TILES
ROWS_PER_TILE = NPAD // 16


@functools.lru_cache(maxsize=None)
def _make_sc_scatter(width):
    mesh = plsc.VectorSubcoreMesh(core_axis_name="c", subcore_axis_name="s")

    @functools.partial(
        pl.kernel,
        out_type=jax.ShapeDtypeStruct((2, NPAD, width), jnp.float32),
        mesh=mesh,
        scratch_types=[
            pltpu.VMEM((NCHUNK, CHUNK), jnp.int32),
            pltpu.VMEM((NCHUNK, CHUNK), jnp.int32),
            pltpu.VMEM((CHUNK, width), jnp.float32),
            pltpu.VMEM((CHUNK, width), jnp.float32),
            pltpu.VMEM((128, width), jnp.float32),
            pltpu.VMEM_SHARED((NPAD, width), jnp.float32),
            pltpu.SemaphoreType.DMA,
            pltpu.SemaphoreType.DMA,
        ],
        compiler_params=pltpu.CompilerParams(use_tc_tiling_on_sc=False),
    )
    def sc_scatter(table_hbm, src_hbm, dst_hbm, out_hbm,
                   sidx, didx, rows0, rows1, zblk, acc, sem0, sem1):
        cid = lax.axis_index("c")
        sid = lax.axis_index("s")
        wid = cid * 16 + sid

        @pl.loop(0, 128)
        def _zero_rows(i):
            @pl.loop(0, width // 16)
            def _zero_lanes(j):
                zblk[i, pl.ds(j * 16, 16)] = jnp.zeros((16,), jnp.float32)

        @pl.loop(0, ROWS_PER_TILE // 128)
        def _zero_acc(r):
            pltpu.sync_copy(zblk, acc.at[pl.ds(sid * ROWS_PER_TILE + r * 128, 128)])

        pltpu.sync_copy(src_hbm.at[wid], sidx)
        pltpu.sync_copy(dst_hbm.at[wid], didx)

        plsc.subcore_barrier()

        pltpu.async_copy(table_hbm.at[sidx.at[0]], rows0, sem0)

        @pl.loop(0, NCHUNK // 2)
        def _edges(it):
            ch0 = it * 2
            ch1 = ch0 + 1
            pltpu.make_async_copy(table_hbm.at[sidx.at[ch0]], rows0, sem0).wait()
            pltpu.async_copy(table_hbm.at[sidx.at[ch1]], rows1, sem1)
            pltpu.sync_copy(rows0, acc.at[didx.at[ch0]], add=True)
            pltpu.make_async_copy(table_hbm.at[sidx.at[ch1]], rows1, sem1).wait()
            nxt = jnp.minimum(ch0 + 2, NCHUNK - 1)
            pltpu.async_copy(table_hbm.at[sidx.at[nxt]], rows0, sem0)
            pltpu.sync_copy(rows1, acc.at[didx.at[ch1]], add=True)

        pltpu.make_async_copy(table_hbm.at[sidx.at[NCHUNK - 1]], rows0, sem0).wait()

        plsc.subcore_barrier()

        @pl.loop(0, ROWS_PER_TILE // 128)
        def _writeout(r):
            row0 = sid * ROWS_PER_TILE + r * 128
            pltpu.sync_copy(acc.at[pl.ds(row0, 128)],
                            out_hbm.at[cid, pl.ds(row0, 128)])

    return sc_scatter


_BR = 1000


def _dot_t(a, w):
    return lax.dot_general(a, w, (((1,), (1,)), ((), ())),
                           preferred_element_type=jnp.float32)


def _tc_pre_body(x_ref, w1l_ref, w1r_ref, b1_ref, table_ref, p1_ref):
    x = x_ref[...]
    y = _dot_t(x, w1l_ref[...])
    col = lax.broadcasted_iota(jnp.int32, (_BR, D1), 1)
    table_ref[...] = y + jnp.where(col == D_H, 1.0, 0.0)
    p1_ref[...] = _dot_t(x, w1r_ref[...]) + b1_ref[...]


def _tc_mid_body(a0_ref, a1_ref, p1_ref, w2l_ref, w2r_ref, b2_ref,
                 table2_ref, p2_ref, dinv_ref):
    acc = a0_ref[...] + a1_ref[...]
    deg = acc[:, D_H:D_H + 1]
    dinv = 1.0 / jnp.maximum(deg, 1.0)
    h = jnp.maximum(acc[:, :D_H] * dinv + p1_ref[...], 0.0)
    table2_ref[...] = _dot_t(h, w2l_ref[...])
    p2_ref[...] = _dot_t(h, w2r_ref[...]) + b2_ref[...]
    dinv_ref[...] = dinv


def _tc_post_body(a0_ref, a1_ref, dinv_ref, p2_ref, out_ref):
    agg = (a0_ref[...] + a1_ref[...])[:, :D_OUT] * dinv_ref[...]
    z = agg + p2_ref[...]
    m = jnp.max(z, axis=1, keepdims=True)
    zs = z - m
    out_ref[...] = zs - jnp.log(jnp.sum(jnp.exp(zs), axis=1, keepdims=True))


def _row_spec(w):
    return pl.BlockSpec((_BR, w), lambda i: (i, 0))


def _full_spec(shape):
    return pl.BlockSpec(shape, lambda i: tuple(0 for _ in shape))


def kernel(x, edge_index, W1l, W1r, b1, W2l, W2r, b2):
    n_fake = E_PAD - E
    src = jnp.concatenate(
        [edge_index[0], jnp.zeros((n_fake,), jnp.int32)]
    ).reshape(NUM_TILES, NCHUNK, CHUNK)
    dst = jnp.concatenate(
        [edge_index[1], jnp.full((n_fake,), NPAD - 1, jnp.int32)]
    ).reshape(NUM_TILES, NCHUNK, CHUNK)

    w1l_pad = jnp.zeros((D1, D_IN), jnp.float32).at[:D_H].set(W1l)
    w2l_pad = jnp.zeros((D2, D_H), jnp.float32).at[:D_OUT].set(W2l)
    b1r = b1.reshape(1, D_H)
    b2r = b2.reshape(1, D_OUT)

    grid = (N // _BR,)

    table1, p1 = pl.pallas_call(
        _tc_pre_body,
        grid=grid,
        in_specs=[_row_spec(D_IN), _full_spec((D1, D_IN)),
                  _full_spec((D_H, D_IN)), _full_spec((1, D_H))],
        out_specs=[_row_spec(D1), _row_spec(D_H)],
        out_shape=[jax.ShapeDtypeStruct((N, D1), jnp.float32),
                   jax.ShapeDtypeStruct((N, D_H), jnp.float32)],
    )(x, w1l_pad, W1r, b1r)

    part1 = _make_sc_scatter(D1)(table1, src, dst)

    table2, p2, dinv = pl.pallas_call(
        _tc_mid_body,
        grid=grid,
        in_specs=[_row_spec(D1), _row_spec(D1), _row_spec(D_H),
                  _full_spec((D2, D_H)), _full_spec((D_OUT, D_H)),
                  _full_spec((1, D_OUT))],
        out_specs=[_row_spec(D2), _row_spec(D_OUT), _row_spec(1)],
        out_shape=[jax.ShapeDtypeStruct((N, D2), jnp.float32),
                   jax.ShapeDtypeStruct((N, D_OUT), jnp.float32),
                   jax.ShapeDtypeStruct((N, 1), jnp.float32)],
    )(part1[0], part1[1], p1, w2l_pad, W2r, b2r)

    part2 = _make_sc_scatter(D2)(table2, src, dst)

    out = pl.pallas_call(
        _tc_post_body,
        grid=grid,
        in_specs=[_row_spec(D2), _row_spec(D2), _row_spec(1),
                  _row_spec(D_OUT)],
        out_specs=_row_spec(D_OUT),
        out_shape=jax.ShapeDtypeStruct((N, D_OUT), jnp.float32),
    )(part2[0], part2[1], dinv, p2)

    return out
